# Initial kernel scaffold; baseline (speedup 1.0000x reference)
import jax
import jax.numpy as jnp
from jax.experimental import pallas as pl


def kernel(x, edge_index, edge_weight, batch, W1, b1, W2, b2, Wm, bm):
    raise NotImplementedError("write your pallas kernel here")


# speedup vs baseline: 12.5091x; 12.5091x over previous
import functools

import jax
import jax.numpy as jnp
from jax import lax
from jax.experimental import pallas as pl
from jax.experimental.pallas import tpu as pltpu
from jax.experimental.pallas import tpu_sc as plsc

N_NODES = 10000
N_EDGES = 320000
N_GRAPHS = 64
D_IN, D_HID, D_OUT = 128, 256, 128

NC, NS = 2, 16
LANES = 16
CHK = 80
ROWS_PER_TILE = N_NODES // NS

_MESH = plsc.VectorSubcoreMesh(
    core_axis_name="c", subcore_axis_name="s", num_cores=NC, num_subcores=NS)


def _relu_inplace(ref, n):
  def body(v, _):
    sl = pl.ds(v * LANES, LANES)
    ref[sl] = jnp.maximum(ref[sl], 0.0)
    return _

  lax.fori_loop(0, n // LANES, body, None)


def _make_deg_kernel():
  epw = N_EDGES // (NC * NS)
  nchk = epw // CHK
  npad = 10240
  rpt = npad // NS

  @functools.partial(
      pl.kernel,
      out_type=jax.ShapeDtypeStruct((NC, npad), jnp.float32),
      mesh=_MESH,
      scratch_types=[
          pltpu.VMEM((epw,), jnp.int32),
          pltpu.VMEM((epw,), jnp.float32),
          pltpu.VMEM((CHK,), jnp.int32),
          pltpu.VMEM((rpt,), jnp.float32),
          pltpu.VMEM_SHARED((npad,), jnp.float32),
      ],
      name="gcn_deg_sc",
  )
  def deg_kernel(col_hbm, ew_hbm, out_hbm, colflat, ewflat, colchunk, zbuf,
                 acc):
    cid = lax.axis_index("c")
    sid = lax.axis_index("s")
    estart = (cid * NS + sid) * epw
    pltpu.sync_copy(col_hbm.at[pl.ds(estart, epw)], colflat)
    pltpu.sync_copy(ew_hbm.at[pl.ds(estart, epw)], ewflat)
    _relu_inplace(ewflat, epw)

    def zrow(r, _):
      zbuf[pl.ds(r * LANES, LANES)] = jnp.zeros((LANES,), jnp.float32)
      return _

    lax.fori_loop(0, rpt // LANES, zrow, None)
    pltpu.sync_copy(zbuf, acc.at[pl.ds(sid * rpt, rpt)])
    plsc.subcore_barrier()

    def chunk(i, _):
      e0 = i * CHK
      for k in range(CHK // LANES):
        colchunk[pl.ds(k * LANES, LANES)] = colflat[pl.ds(e0 + k * LANES,
                                                          LANES)]
      pltpu.sync_copy(ewflat.at[pl.ds(e0, CHK)], acc.at[colchunk], add=True)
      return _

    lax.fori_loop(0, nchk, chunk, None)
    plsc.subcore_barrier()
    pltpu.sync_copy(acc.at[pl.ds(sid * rpt, rpt)],
                    out_hbm.at[cid, pl.ds(sid * rpt, rpt)])
    return None

  return deg_kernel


def _make_spmm_kernel(split_edges, ntab):
  epw = N_EDGES // (NS * (NC if split_edges else 1))
  blk = 2000
  npad = 10240
  rpt = npad // NS

  @functools.partial(
      pl.kernel,
      out_type=jax.ShapeDtypeStruct((NC, npad, D_OUT), jnp.float32),
      mesh=_MESH,
      scratch_types=[
          pltpu.VMEM((blk,), jnp.int32),
          pltpu.VMEM((blk,), jnp.int32),
          pltpu.VMEM((blk,), jnp.float32),
          pltpu.VMEM((CHK,), jnp.int32),
          pltpu.VMEM((CHK, D_OUT), jnp.float32),
          pltpu.VMEM_SHARED((npad, D_OUT), jnp.float32),
          pltpu.SemaphoreType.DMA,
      ],
      name="gcn_spmm_sc",
  )
  def spmm_kernel(row_hbm, col_hbm, ew_hbm, tbl_hbm, out_hbm, rowblk,
                  colblk, ewblk, colchunk, gbuf, acc, sem):
    cid = lax.axis_index("c")
    sid = lax.axis_index("s")
    if split_edges:
      estart = (cid * NS + sid) * epw
    else:
      estart = sid * epw

    def zrow(r, _):
      for k in range(D_OUT // LANES):
        gbuf[r, pl.ds(k * LANES, LANES)] = jnp.zeros((LANES,), jnp.float32)
      return _

    lax.fori_loop(0, CHK, zrow, None)
    for k in range(rpt // CHK):
      pltpu.sync_copy(gbuf, acc.at[pl.ds(sid * rpt + k * CHK, CHK)])
    plsc.subcore_barrier()

    def block(bi, _):
      eb = estart + bi * blk
      pltpu.sync_copy(row_hbm.at[pl.ds(eb, blk)], rowblk)
      pltpu.sync_copy(col_hbm.at[pl.ds(eb, blk)], colblk)
      pltpu.sync_copy(ew_hbm.at[pl.ds(eb, blk)], ewblk)

      if ntab == 2:
        def addoff(v, c):
          sl = pl.ds(v * LANES, LANES)
          rowblk[sl] = rowblk[sl] + cid * N_NODES
          return c

        lax.fori_loop(0, blk // LANES, addoff, None)

      def chunk(ci, c):
        e0 = ci * CHK
        for k in range(CHK // LANES):
          colchunk[pl.ds(k * LANES, LANES)] = colblk[pl.ds(e0 + k * LANES,
                                                           LANES)]
        pltpu.async_copy(tbl_hbm.at[rowblk.at[pl.ds(e0, CHK)]], gbuf,
                         sem).wait()

        def rowgrp(jb, cc):
          ewv = jnp.maximum(ewblk[pl.ds(e0 + jb * LANES, LANES)], 0.0)
          for j2 in range(LANES):
            sv = jnp.full((LANES,), ewv[j2], jnp.float32)
            j = jb * LANES + j2
            for k in range(D_OUT // LANES):
              sl = pl.ds(k * LANES, LANES)
              gbuf[j, sl] = gbuf[j, sl] * sv
          return cc

        lax.fori_loop(0, CHK // LANES, rowgrp, None)
        pltpu.sync_copy(gbuf, acc.at[colchunk], add=True)
        return c

      lax.fori_loop(0, blk // CHK, chunk, None)
      return _

    lax.fori_loop(0, epw // blk, block, None)
    plsc.subcore_barrier()
    pltpu.sync_copy(acc.at[pl.ds(sid * rpt, rpt)],
                    out_hbm.at[cid, pl.ds(sid * rpt, rpt)])
    return None

  return spmm_kernel


_deg_kernel = _make_deg_kernel()
_spmm_l1 = _make_spmm_kernel(split_edges=False, ntab=2)
_spmm_l2 = _make_spmm_kernel(split_edges=True, ntab=1)

NB = 1000
NBLK = N_NODES // NB


def _dis_block(degp):
  deg = degp[:, 0:1] + degp[:, 1:2] + 1.0
  return lax.rsqrt(deg)


def _tca_body(degp_ref, x_ref, w1_ref, g_ref):
  dis = _dis_block(degp_ref[...])
  xs = x_ref[...] * dis
  g_ref[0] = jnp.dot(xs, w1_ref[:, 0:128], preferred_element_type=jnp.float32)
  g_ref[1] = jnp.dot(xs, w1_ref[:, 128:256],
                     preferred_element_type=jnp.float32)


def _tcb_body(degp_ref, s1_ref, g1_ref, b1_ref, w2_ref, g2_ref):
  dis = _dis_block(degp_ref[...])
  hr0 = jnp.maximum((s1_ref[0] + g1_ref[0]) * dis + b1_ref[0:1, :], 0.0)
  hr1 = jnp.maximum((s1_ref[1] + g1_ref[1]) * dis + b1_ref[1:2, :], 0.0)
  h2 = (jnp.dot(hr0, w2_ref[0], preferred_element_type=jnp.float32) +
        jnp.dot(hr1, w2_ref[1], preferred_element_type=jnp.float32))
  g2_ref[...] = h2 * dis


def _tcc_body(degp_ref, s2_ref, g2_ref, b2_ref, batch_ref, wm_ref, bm_ref,
              out_ref, pooled, counts):
  i = pl.program_id(0)
  dis = _dis_block(degp_ref[...])
  hf = jnp.maximum((s2_ref[0] + s2_ref[1] + g2_ref[...]) * dis +
                   b2_ref[0:1, :], 0.0)
  onehot = (batch_ref[...] == lax.broadcasted_iota(
      jnp.int32, (1, N_GRAPHS), 1)).astype(jnp.float32)
  psum = lax.dot_general(onehot, hf, (((0,), (0,)), ((), ())),
                         preferred_element_type=jnp.float32)
  csum = lax.dot_general(onehot, jnp.ones((NB, 1), jnp.float32),
                         (((0,), (0,)), ((), ())),
                         preferred_element_type=jnp.float32)

  @pl.when(i == 0)
  def _():
    pooled[...] = jnp.zeros_like(pooled)
    counts[...] = jnp.zeros_like(counts)

  pooled[...] += psum
  counts[...] += csum
  mean = pooled[...] / jnp.maximum(counts[...], 1.0)
  out_ref[...] = jnp.dot(mean, wm_ref[...],
                         preferred_element_type=jnp.float32) + bm_ref[...]


def kernel(x, edge_index, edge_weight, batch, W1, b1, W2, b2, Wm, bm):
  row = edge_index[0].astype(jnp.int32)
  col = edge_index[1].astype(jnp.int32)
  ew = edge_weight.astype(jnp.float32)

  degp = _deg_kernel(col, ew)[:, :N_NODES]
  degp_t = degp.T

  g1 = pl.pallas_call(
      _tca_body,
      grid=(NBLK,),
      in_specs=[
          pl.BlockSpec((NB, 2), lambda i: (i, 0)),
          pl.BlockSpec((NB, D_IN), lambda i: (i, 0)),
          pl.BlockSpec((D_IN, D_HID), lambda i: (0, 0)),
      ],
      out_specs=pl.BlockSpec((2, NB, 128), lambda i: (0, i, 0)),
      out_shape=jax.ShapeDtypeStruct((2, N_NODES, 128), jnp.float32),
      name="gcn_tca",
  )(degp_t, x, W1)

  s1 = _spmm_l1(row, col, ew, g1.reshape(2 * N_NODES, 128))[:, :N_NODES]

  b1r = b1.reshape(2, 128)
  w2r = W2.reshape(2, 128, D_OUT)
  g2 = pl.pallas_call(
      _tcb_body,
      grid=(NBLK,),
      in_specs=[
          pl.BlockSpec((NB, 2), lambda i: (i, 0)),
          pl.BlockSpec((2, NB, 128), lambda i: (0, i, 0)),
          pl.BlockSpec((2, NB, 128), lambda i: (0, i, 0)),
          pl.BlockSpec((2, 128), lambda i: (0, 0)),
          pl.BlockSpec((2, 128, D_OUT), lambda i: (0, 0, 0)),
      ],
      out_specs=pl.BlockSpec((NB, D_OUT), lambda i: (i, 0)),
      out_shape=jax.ShapeDtypeStruct((N_NODES, D_OUT), jnp.float32),
      name="gcn_tcb",
  )(degp_t, s1, g1, b1r, w2r)

  s2 = _spmm_l2(row, col, ew, g2)[:, :N_NODES]

  out = pl.pallas_call(
      _tcc_body,
      grid=(NBLK,),
      in_specs=[
          pl.BlockSpec((NB, 2), lambda i: (i, 0)),
          pl.BlockSpec((2, NB, D_OUT), lambda i: (0, i, 0)),
          pl.BlockSpec((NB, D_OUT), lambda i: (i, 0)),
          pl.BlockSpec((1, D_OUT), lambda i: (0, 0)),
          pl.BlockSpec((NB, 1), lambda i: (i, 0)),
          pl.BlockSpec((D_OUT, 2), lambda i: (0, 0)),
          pl.BlockSpec((1, 2), lambda i: (0, 0)),
      ],
      out_specs=pl.BlockSpec((N_GRAPHS, 2), lambda i: (0, 0)),
      out_shape=jax.ShapeDtypeStruct((N_GRAPHS, 2), jnp.float32),
      scratch_shapes=[
          pltpu.VMEM((N_GRAPHS, D_OUT), jnp.float32),
          pltpu.VMEM((N_GRAPHS, 1), jnp.float32),
      ],
      name="gcn_tcc",
  )(degp_t, s2, g2, b2.reshape(1, D_OUT), batch.astype(jnp.int32)[:, None],
    Wm, bm.reshape(1, 2))
  return out


# speedup vs baseline: 19.7112x; 1.5758x over previous
import functools

import jax
import jax.numpy as jnp
from jax import lax
from jax.experimental import pallas as pl
from jax.experimental.pallas import tpu as pltpu
from jax.experimental.pallas import tpu_sc as plsc

N_NODES = 10000
N_EDGES = 320000
N_GRAPHS = 64
D_IN, D_HID, D_OUT = 128, 256, 128

NC, NS = 2, 16
LANES = 16
CHK = 80
ROWS_PER_TILE = N_NODES // NS

_MESH = plsc.VectorSubcoreMesh(
    core_axis_name="c", subcore_axis_name="s", num_cores=NC, num_subcores=NS)


def _relu_inplace(ref, n):
  def body(v, _):
    sl = pl.ds(v * LANES, LANES)
    ref[sl] = jnp.maximum(ref[sl], 0.0)
    return _

  lax.fori_loop(0, n // LANES, body, None)


def _make_deg_kernel():
  epw = N_EDGES // (NC * NS)
  nchk = epw // CHK
  npad = 10240
  rpt = npad // NS

  @functools.partial(
      pl.kernel,
      out_type=jax.ShapeDtypeStruct((NC, npad), jnp.float32),
      mesh=_MESH,
      scratch_types=[
          pltpu.VMEM((epw,), jnp.int32),
          pltpu.VMEM((epw,), jnp.float32),
          pltpu.VMEM((CHK,), jnp.int32),
          pltpu.VMEM((rpt,), jnp.float32),
          pltpu.VMEM_SHARED((npad,), jnp.float32),
      ],
      name="gcn_deg_sc",
  )
  def deg_kernel(col_hbm, ew_hbm, out_hbm, colflat, ewflat, colchunk, zbuf,
                 acc):
    cid = lax.axis_index("c")
    sid = lax.axis_index("s")
    estart = (cid * NS + sid) * epw
    pltpu.sync_copy(col_hbm.at[pl.ds(estart, epw)], colflat)
    pltpu.sync_copy(ew_hbm.at[pl.ds(estart, epw)], ewflat)
    _relu_inplace(ewflat, epw)

    def zrow(r, _):
      zbuf[pl.ds(r * LANES, LANES)] = jnp.zeros((LANES,), jnp.float32)
      return _

    lax.fori_loop(0, rpt // LANES, zrow, None)
    pltpu.sync_copy(zbuf, acc.at[pl.ds(sid * rpt, rpt)])
    plsc.subcore_barrier()

    def chunk(i, _):
      e0 = i * CHK
      for k in range(CHK // LANES):
        colchunk[pl.ds(k * LANES, LANES)] = colflat[pl.ds(e0 + k * LANES,
                                                          LANES)]
      pltpu.sync_copy(ewflat.at[pl.ds(e0, CHK)], acc.at[colchunk], add=True)
      return _

    lax.fori_loop(0, nchk, chunk, None)
    plsc.subcore_barrier()
    pltpu.sync_copy(acc.at[pl.ds(sid * rpt, rpt)],
                    out_hbm.at[cid, pl.ds(sid * rpt, rpt)])
    return None

  return deg_kernel


def _make_spmm_kernel(split_edges, ntab):
  epw = N_EDGES // (NS * (NC if split_edges else 1))
  blk = 2000
  npad = 10240
  rpt = npad // NS

  @functools.partial(
      pl.kernel,
      out_type=jax.ShapeDtypeStruct((NC, npad, D_OUT), jnp.float32),
      mesh=_MESH,
      scratch_types=[
          pltpu.VMEM((blk,), jnp.int32),
          pltpu.VMEM((blk,), jnp.int32),
          pltpu.VMEM((blk,), jnp.float32),
          pltpu.VMEM((CHK,), jnp.int32),
          pltpu.VMEM((CHK, D_OUT), jnp.float32),
          pltpu.VMEM((CHK, D_OUT), jnp.float32),
          pltpu.VMEM_SHARED((npad, D_OUT), jnp.float32),
          pltpu.SemaphoreType.DMA,
          pltpu.SemaphoreType.DMA,
      ],
      name="gcn_spmm_sc",
  )
  def spmm_kernel(row_hbm, col_hbm, ew_hbm, tbl_hbm, out_hbm, rowblk,
                  colblk, ewblk, colchunk, gbuf0, gbuf1, acc, sem0, sem1):
    cid = lax.axis_index("c")
    sid = lax.axis_index("s")
    if split_edges:
      estart = (cid * NS + sid) * epw
    else:
      estart = sid * epw

    def zrow(r, _):
      for k in range(D_OUT // LANES):
        gbuf0[r, pl.ds(k * LANES, LANES)] = jnp.zeros((LANES,), jnp.float32)
      return _

    lax.fori_loop(0, CHK, zrow, None)
    for k in range(rpt // CHK):
      pltpu.sync_copy(gbuf0, acc.at[pl.ds(sid * rpt + k * CHK, CHK)])
    plsc.subcore_barrier()

    nchunk = blk // CHK

    def gather(ci, gbuf, sem):
      return pltpu.async_copy(tbl_hbm.at[rowblk.at[pl.ds(ci * CHK, CHK)]],
                              gbuf, sem)

    def process(ci, gbuf):
      e0 = ci * CHK
      for k in range(CHK // LANES):
        colchunk[pl.ds(k * LANES, LANES)] = colblk[pl.ds(e0 + k * LANES,
                                                         LANES)]

      def rowgrp(jb, cc):
        ewv = jnp.maximum(ewblk[pl.ds(e0 + jb * LANES, LANES)], 0.0)
        for j2 in range(LANES):
          sv = jnp.full((LANES,), ewv[j2], jnp.float32)
          j = jb * LANES + j2
          for k in range(D_OUT // LANES):
            sl = pl.ds(k * LANES, LANES)
            gbuf[j, sl] = gbuf[j, sl] * sv
        return cc

      lax.fori_loop(0, CHK // LANES, rowgrp, None)
      pltpu.sync_copy(gbuf, acc.at[colchunk], add=True)

    def block(bi, _):
      eb = estart + bi * blk
      pltpu.sync_copy(row_hbm.at[pl.ds(eb, blk)], rowblk)
      pltpu.sync_copy(col_hbm.at[pl.ds(eb, blk)], colblk)
      pltpu.sync_copy(ew_hbm.at[pl.ds(eb, blk)], ewblk)

      if ntab == 2:
        def addoff(v, c):
          sl = pl.ds(v * LANES, LANES)
          rowblk[sl] = rowblk[sl] + cid * N_NODES
          return c

        lax.fori_loop(0, blk // LANES, addoff, None)

      gather(0, gbuf0, sem0).wait()

      def pair_body(pi, c):
        c0 = pi * 2
        g1 = gather(c0 + 1, gbuf1, sem1)
        process(c0, gbuf0)
        g0 = gather(jnp.minimum(c0 + 2, nchunk - 1), gbuf0, sem0)
        g1.wait()
        process(c0 + 1, gbuf1)
        g0.wait()
        return c

      lax.fori_loop(0, nchunk // 2, pair_body, None)
      if nchunk % 2 == 1:
        process(nchunk - 1, gbuf0)
      return _

    lax.fori_loop(0, epw // blk, block, None)
    plsc.subcore_barrier()
    pltpu.sync_copy(acc.at[pl.ds(sid * rpt, rpt)],
                    out_hbm.at[cid, pl.ds(sid * rpt, rpt)])
    return None

  return spmm_kernel


_deg_kernel = _make_deg_kernel()
_spmm_l1 = _make_spmm_kernel(split_edges=False, ntab=2)
_spmm_l2 = _make_spmm_kernel(split_edges=True, ntab=1)

NB = 1000
NBLK = N_NODES // NB


def _dis_block(degp):
  deg = degp[:, 0:1] + degp[:, 1:2] + 1.0
  return lax.rsqrt(deg)


def _tca_body(degp_ref, x_ref, w1_ref, g_ref):
  dis = _dis_block(degp_ref[...])
  xs = x_ref[...] * dis
  g_ref[0] = jnp.dot(xs, w1_ref[:, 0:128], preferred_element_type=jnp.float32)
  g_ref[1] = jnp.dot(xs, w1_ref[:, 128:256],
                     preferred_element_type=jnp.float32)


def _tcb_body(degp_ref, s1_ref, g1_ref, b1_ref, w2_ref, g2_ref):
  dis = _dis_block(degp_ref[...])
  hr0 = jnp.maximum((s1_ref[0] + g1_ref[0]) * dis + b1_ref[0:1, :], 0.0)
  hr1 = jnp.maximum((s1_ref[1] + g1_ref[1]) * dis + b1_ref[1:2, :], 0.0)
  h2 = (jnp.dot(hr0, w2_ref[0], preferred_element_type=jnp.float32) +
        jnp.dot(hr1, w2_ref[1], preferred_element_type=jnp.float32))
  g2_ref[...] = h2 * dis


def _tcc_body(degp_ref, s2_ref, g2_ref, b2_ref, batch_ref, wm_ref, bm_ref,
              out_ref, pooled, counts):
  i = pl.program_id(0)
  dis = _dis_block(degp_ref[...])
  hf = jnp.maximum((s2_ref[0] + s2_ref[1] + g2_ref[...]) * dis +
                   b2_ref[0:1, :], 0.0)
  onehot = (batch_ref[...] == lax.broadcasted_iota(
      jnp.int32, (1, N_GRAPHS), 1)).astype(jnp.float32)
  psum = lax.dot_general(onehot, hf, (((0,), (0,)), ((), ())),
                         preferred_element_type=jnp.float32)
  csum = lax.dot_general(onehot, jnp.ones((NB, 1), jnp.float32),
                         (((0,), (0,)), ((), ())),
                         preferred_element_type=jnp.float32)

  @pl.when(i == 0)
  def _():
    pooled[...] = jnp.zeros_like(pooled)
    counts[...] = jnp.zeros_like(counts)

  pooled[...] += psum
  counts[...] += csum
  mean = pooled[...] / jnp.maximum(counts[...], 1.0)
  out_ref[...] = jnp.dot(mean, wm_ref[...],
                         preferred_element_type=jnp.float32) + bm_ref[...]


def kernel(x, edge_index, edge_weight, batch, W1, b1, W2, b2, Wm, bm):
  row = edge_index[0].astype(jnp.int32)
  col = edge_index[1].astype(jnp.int32)
  ew = edge_weight.astype(jnp.float32)

  degp = _deg_kernel(col, ew)[:, :N_NODES]
  degp_t = degp.T

  g1 = pl.pallas_call(
      _tca_body,
      grid=(NBLK,),
      in_specs=[
          pl.BlockSpec((NB, 2), lambda i: (i, 0)),
          pl.BlockSpec((NB, D_IN), lambda i: (i, 0)),
          pl.BlockSpec((D_IN, D_HID), lambda i: (0, 0)),
      ],
      out_specs=pl.BlockSpec((2, NB, 128), lambda i: (0, i, 0)),
      out_shape=jax.ShapeDtypeStruct((2, N_NODES, 128), jnp.float32),
      name="gcn_tca",
  )(degp_t, x, W1)

  s1 = _spmm_l1(row, col, ew, g1.reshape(2 * N_NODES, 128))[:, :N_NODES]

  b1r = b1.reshape(2, 128)
  w2r = W2.reshape(2, 128, D_OUT)
  g2 = pl.pallas_call(
      _tcb_body,
      grid=(NBLK,),
      in_specs=[
          pl.BlockSpec((NB, 2), lambda i: (i, 0)),
          pl.BlockSpec((2, NB, 128), lambda i: (0, i, 0)),
          pl.BlockSpec((2, NB, 128), lambda i: (0, i, 0)),
          pl.BlockSpec((2, 128), lambda i: (0, 0)),
          pl.BlockSpec((2, 128, D_OUT), lambda i: (0, 0, 0)),
      ],
      out_specs=pl.BlockSpec((NB, D_OUT), lambda i: (i, 0)),
      out_shape=jax.ShapeDtypeStruct((N_NODES, D_OUT), jnp.float32),
      name="gcn_tcb",
  )(degp_t, s1, g1, b1r, w2r)

  s2 = _spmm_l2(row, col, ew, g2)[:, :N_NODES]

  out = pl.pallas_call(
      _tcc_body,
      grid=(NBLK,),
      in_specs=[
          pl.BlockSpec((NB, 2), lambda i: (i, 0)),
          pl.BlockSpec((2, NB, D_OUT), lambda i: (0, i, 0)),
          pl.BlockSpec((NB, D_OUT), lambda i: (i, 0)),
          pl.BlockSpec((1, D_OUT), lambda i: (0, 0)),
          pl.BlockSpec((NB, 1), lambda i: (i, 0)),
          pl.BlockSpec((D_OUT, 2), lambda i: (0, 0)),
          pl.BlockSpec((1, 2), lambda i: (0, 0)),
      ],
      out_specs=pl.BlockSpec((N_GRAPHS, 2), lambda i: (0, 0)),
      out_shape=jax.ShapeDtypeStruct((N_GRAPHS, 2), jnp.float32),
      scratch_shapes=[
          pltpu.VMEM((N_GRAPHS, D_OUT), jnp.float32),
          pltpu.VMEM((N_GRAPHS, 1), jnp.float32),
      ],
      name="gcn_tcc",
  )(degp_t, s2, g2, b2.reshape(1, D_OUT), batch.astype(jnp.int32)[:, None],
    Wm, bm.reshape(1, 2))
  return out


# speedup vs baseline: 22.1735x; 1.1249x over previous
import functools

import jax
import jax.numpy as jnp
from jax import lax
from jax.experimental import pallas as pl
from jax.experimental.pallas import tpu as pltpu
from jax.experimental.pallas import tpu_sc as plsc

N_NODES = 10000
N_EDGES = 320000
N_GRAPHS = 64
D_IN, D_HID, D_OUT = 128, 256, 128

NC, NS = 2, 16
LANES = 16
CHK = 80
ROWS_PER_TILE = N_NODES // NS

_MESH = plsc.VectorSubcoreMesh(
    core_axis_name="c", subcore_axis_name="s", num_cores=NC, num_subcores=NS)


def _relu_inplace(ref, n):
  def body(v, _):
    sl = pl.ds(v * LANES, LANES)
    ref[sl] = jnp.maximum(ref[sl], 0.0)
    return _

  lax.fori_loop(0, n // LANES, body, None)


def _make_deg_kernel():
  epw = N_EDGES // (NC * NS)
  nchk = epw // CHK
  npad = 10240
  rpt = npad // NS

  @functools.partial(
      pl.kernel,
      out_type=jax.ShapeDtypeStruct((NC, npad), jnp.float32),
      mesh=_MESH,
      scratch_types=[
          pltpu.VMEM((epw,), jnp.int32),
          pltpu.VMEM((epw,), jnp.float32),
          pltpu.VMEM((CHK,), jnp.int32),
          pltpu.VMEM((rpt,), jnp.float32),
          pltpu.VMEM_SHARED((npad,), jnp.float32),
      ],
      name="gcn_deg_sc",
  )
  def deg_kernel(col_hbm, ew_hbm, out_hbm, colflat, ewflat, colchunk, zbuf,
                 acc):
    cid = lax.axis_index("c")
    sid = lax.axis_index("s")
    estart = (cid * NS + sid) * epw
    pltpu.sync_copy(col_hbm.at[pl.ds(estart, epw)], colflat)
    pltpu.sync_copy(ew_hbm.at[pl.ds(estart, epw)], ewflat)
    _relu_inplace(ewflat, epw)

    def zrow(r, _):
      zbuf[pl.ds(r * LANES, LANES)] = jnp.zeros((LANES,), jnp.float32)
      return _

    lax.fori_loop(0, rpt // LANES, zrow, None)
    pltpu.sync_copy(zbuf, acc.at[pl.ds(sid * rpt, rpt)])
    plsc.subcore_barrier()

    def chunk(i, _):
      e0 = i * CHK
      for k in range(CHK // LANES):
        colchunk[pl.ds(k * LANES, LANES)] = colflat[pl.ds(e0 + k * LANES,
                                                          LANES)]
      pltpu.sync_copy(ewflat.at[pl.ds(e0, CHK)], acc.at[colchunk], add=True)
      return _

    lax.fori_loop(0, nchk, chunk, None)
    plsc.subcore_barrier()
    pltpu.sync_copy(acc.at[pl.ds(sid * rpt, rpt)],
                    out_hbm.at[cid, pl.ds(sid * rpt, rpt)])
    return None

  return deg_kernel


def _make_spmm_kernel(split_edges, ntab):
  epw = N_EDGES // (NS * (NC if split_edges else 1))
  blk = 2000
  npad = 10240
  rpt = npad // NS

  @functools.partial(
      pl.kernel,
      out_type=jax.ShapeDtypeStruct((NC, npad, D_OUT), jnp.float32),
      mesh=_MESH,
      scratch_types=[
          pltpu.VMEM((blk,), jnp.int32),
          pltpu.VMEM((blk,), jnp.int32),
          pltpu.VMEM((blk,), jnp.float32),
          pltpu.VMEM((CHK,), jnp.int32),
          pltpu.VMEM((CHK,), jnp.int32),
          pltpu.VMEM((CHK,), jnp.int32),
          pltpu.VMEM((CHK, D_OUT), jnp.float32),
          pltpu.VMEM((CHK, D_OUT), jnp.float32),
          pltpu.VMEM((CHK, D_OUT), jnp.float32),
          pltpu.VMEM_SHARED((npad, D_OUT), jnp.float32),
          pltpu.SemaphoreType.DMA,
          pltpu.SemaphoreType.DMA,
          pltpu.SemaphoreType.DMA,
          pltpu.SemaphoreType.DMA,
          pltpu.SemaphoreType.DMA,
          pltpu.SemaphoreType.DMA,
      ],
      name="gcn_spmm_sc",
  )
  def spmm_kernel(row_hbm, col_hbm, ew_hbm, tbl_hbm, out_hbm, rowblk,
                  colblk, ewblk, cc0, cc1, cc2, gbuf0, gbuf1, gbuf2, acc,
                  gs0, gs1, gs2, ss0, ss1, ss2):
    cid = lax.axis_index("c")
    sid = lax.axis_index("s")
    if split_edges:
      estart = (cid * NS + sid) * epw
    else:
      estart = sid * epw

    def zrow(r, _):
      for k in range(D_OUT // LANES):
        gbuf0[r, pl.ds(k * LANES, LANES)] = jnp.zeros((LANES,), jnp.float32)
      return _

    lax.fori_loop(0, CHK, zrow, None)
    for k in range(rpt // CHK):
      pltpu.sync_copy(gbuf0, acc.at[pl.ds(sid * rpt + k * CHK, CHK)])
    plsc.subcore_barrier()

    nchunk = blk // CHK
    assert nchunk % 3 == 1
    last = nchunk - 1

    def gather(ci, gbuf, sem):
      ci = jnp.minimum(ci, last)
      return pltpu.async_copy(tbl_hbm.at[rowblk.at[pl.ds(ci * CHK, CHK)]],
                              gbuf, sem)

    def scale(ci, gbuf, cc):
      e0 = ci * CHK
      for k in range(CHK // LANES):
        cc[pl.ds(k * LANES, LANES)] = colblk[pl.ds(e0 + k * LANES, LANES)]

      def rowgrp(jb, c):
        ewv = jnp.maximum(ewblk[pl.ds(e0 + jb * LANES, LANES)], 0.0)
        for j2 in range(LANES):
          sv = jnp.full((LANES,), ewv[j2], jnp.float32)
          j = jb * LANES + j2
          for k in range(D_OUT // LANES):
            sl = pl.ds(k * LANES, LANES)
            gbuf[j, sl] = gbuf[j, sl] * sv
        return c

      lax.fori_loop(0, CHK // LANES, rowgrp, None)

    def scatter(gbuf, cc, sem):
      return pltpu.async_copy(gbuf, acc.at[cc], sem, add=True)

    def block(bi, _):
      eb = estart + bi * blk
      pltpu.sync_copy(row_hbm.at[pl.ds(eb, blk)], rowblk)
      pltpu.sync_copy(col_hbm.at[pl.ds(eb, blk)], colblk)
      pltpu.sync_copy(ew_hbm.at[pl.ds(eb, blk)], ewblk)

      if ntab == 2:
        def addoff(v, c):
          sl = pl.ds(v * LANES, LANES)
          rowblk[sl] = rowblk[sl] + cid * N_NODES
          return c

        lax.fori_loop(0, blk // LANES, addoff, None)

      g1 = gather(0, gbuf0, gs0)
      gather(1, gbuf1, gs1)
      g1.wait()

      def triple(ti, c):
        c0 = ti * 3
        g2 = gather(c0 + 2, gbuf2, gs2)
        scale(c0, gbuf0, cc0)
        s0 = scatter(gbuf0, cc0, ss0)
        pltpu.make_async_copy(
            tbl_hbm.at[rowblk.at[pl.ds(0, CHK)]], gbuf1, gs1).wait()
        scale(c0 + 1, gbuf1, cc1)
        s1 = scatter(gbuf1, cc1, ss1)
        s0.wait()
        g0 = gather(c0 + 3, gbuf0, gs0)
        g2.wait()
        scale(c0 + 2, gbuf2, cc2)
        s2 = scatter(gbuf2, cc2, ss2)
        s1.wait()
        gather(c0 + 4, gbuf1, gs1)
        g0.wait()
        s2.wait()
        return c

      lax.fori_loop(0, nchunk // 3, triple, None)
      scale(last, gbuf0, cc0)
      pltpu.sync_copy(gbuf0, acc.at[cc0], add=True)
      pltpu.make_async_copy(
          tbl_hbm.at[rowblk.at[pl.ds(0, CHK)]], gbuf1, gs1).wait()
      return _

    lax.fori_loop(0, epw // blk, block, None)
    plsc.subcore_barrier()
    pltpu.sync_copy(acc.at[pl.ds(sid * rpt, rpt)],
                    out_hbm.at[cid, pl.ds(sid * rpt, rpt)])
    return None

  return spmm_kernel


_deg_kernel = _make_deg_kernel()
_spmm_l1 = _make_spmm_kernel(split_edges=False, ntab=2)
_spmm_l2 = _make_spmm_kernel(split_edges=True, ntab=1)

NB = 1000
NBLK = N_NODES // NB


def _dis_block(degp):
  deg = degp[:, 0:1] + degp[:, 1:2] + 1.0
  return lax.rsqrt(deg)


def _tca_body(degp_ref, x_ref, w1_ref, g_ref):
  dis = _dis_block(degp_ref[...])
  xs = x_ref[...] * dis
  g_ref[0] = jnp.dot(xs, w1_ref[:, 0:128], preferred_element_type=jnp.float32)
  g_ref[1] = jnp.dot(xs, w1_ref[:, 128:256],
                     preferred_element_type=jnp.float32)


def _tcb_body(degp_ref, s1_ref, g1_ref, b1_ref, w2_ref, g2_ref):
  dis = _dis_block(degp_ref[...])
  hr0 = jnp.maximum((s1_ref[0] + g1_ref[0]) * dis + b1_ref[0:1, :], 0.0)
  hr1 = jnp.maximum((s1_ref[1] + g1_ref[1]) * dis + b1_ref[1:2, :], 0.0)
  h2 = (jnp.dot(hr0, w2_ref[0], preferred_element_type=jnp.float32) +
        jnp.dot(hr1, w2_ref[1], preferred_element_type=jnp.float32))
  g2_ref[...] = h2 * dis


def _tcc_body(degp_ref, s2_ref, g2_ref, b2_ref, batch_ref, wm_ref, bm_ref,
              out_ref, pooled, counts):
  i = pl.program_id(0)
  dis = _dis_block(degp_ref[...])
  hf = jnp.maximum((s2_ref[0] + s2_ref[1] + g2_ref[...]) * dis +
                   b2_ref[0:1, :], 0.0)
  onehot = (batch_ref[...] == lax.broadcasted_iota(
      jnp.int32, (1, N_GRAPHS), 1)).astype(jnp.float32)
  psum = lax.dot_general(onehot, hf, (((0,), (0,)), ((), ())),
                         preferred_element_type=jnp.float32)
  csum = lax.dot_general(onehot, jnp.ones((NB, 1), jnp.float32),
                         (((0,), (0,)), ((), ())),
                         preferred_element_type=jnp.float32)

  @pl.when(i == 0)
  def _():
    pooled[...] = jnp.zeros_like(pooled)
    counts[...] = jnp.zeros_like(counts)

  pooled[...] += psum
  counts[...] += csum
  mean = pooled[...] / jnp.maximum(counts[...], 1.0)
  out_ref[...] = jnp.dot(mean, wm_ref[...],
                         preferred_element_type=jnp.float32) + bm_ref[...]


def kernel(x, edge_index, edge_weight, batch, W1, b1, W2, b2, Wm, bm):
  row = edge_index[0].astype(jnp.int32)
  col = edge_index[1].astype(jnp.int32)
  ew = edge_weight.astype(jnp.float32)

  degp = _deg_kernel(col, ew)[:, :N_NODES]
  degp_t = degp.T

  g1 = pl.pallas_call(
      _tca_body,
      grid=(NBLK,),
      in_specs=[
          pl.BlockSpec((NB, 2), lambda i: (i, 0)),
          pl.BlockSpec((NB, D_IN), lambda i: (i, 0)),
          pl.BlockSpec((D_IN, D_HID), lambda i: (0, 0)),
      ],
      out_specs=pl.BlockSpec((2, NB, 128), lambda i: (0, i, 0)),
      out_shape=jax.ShapeDtypeStruct((2, N_NODES, 128), jnp.float32),
      name="gcn_tca",
  )(degp_t, x, W1)

  s1 = _spmm_l1(row, col, ew, g1.reshape(2 * N_NODES, 128))[:, :N_NODES]

  b1r = b1.reshape(2, 128)
  w2r = W2.reshape(2, 128, D_OUT)
  g2 = pl.pallas_call(
      _tcb_body,
      grid=(NBLK,),
      in_specs=[
          pl.BlockSpec((NB, 2), lambda i: (i, 0)),
          pl.BlockSpec((2, NB, 128), lambda i: (0, i, 0)),
          pl.BlockSpec((2, NB, 128), lambda i: (0, i, 0)),
          pl.BlockSpec((2, 128), lambda i: (0, 0)),
          pl.BlockSpec((2, 128, D_OUT), lambda i: (0, 0, 0)),
      ],
      out_specs=pl.BlockSpec((NB, D_OUT), lambda i: (i, 0)),
      out_shape=jax.ShapeDtypeStruct((N_NODES, D_OUT), jnp.float32),
      name="gcn_tcb",
  )(degp_t, s1, g1, b1r, w2r)

  s2 = _spmm_l2(row, col, ew, g2)[:, :N_NODES]

  out = pl.pallas_call(
      _tcc_body,
      grid=(NBLK,),
      in_specs=[
          pl.BlockSpec((NB, 2), lambda i: (i, 0)),
          pl.BlockSpec((2, NB, D_OUT), lambda i: (0, i, 0)),
          pl.BlockSpec((NB, D_OUT), lambda i: (i, 0)),
          pl.BlockSpec((1, D_OUT), lambda i: (0, 0)),
          pl.BlockSpec((NB, 1), lambda i: (i, 0)),
          pl.BlockSpec((D_OUT, 2), lambda i: (0, 0)),
          pl.BlockSpec((1, 2), lambda i: (0, 0)),
      ],
      out_specs=pl.BlockSpec((N_GRAPHS, 2), lambda i: (0, 0)),
      out_shape=jax.ShapeDtypeStruct((N_GRAPHS, 2), jnp.float32),
      scratch_shapes=[
          pltpu.VMEM((N_GRAPHS, D_OUT), jnp.float32),
          pltpu.VMEM((N_GRAPHS, 1), jnp.float32),
      ],
      name="gcn_tcc",
  )(degp_t, s2, g2, b2.reshape(1, D_OUT), batch.astype(jnp.int32)[:, None],
    Wm, bm.reshape(1, 2))
  return out


# speedup vs baseline: 29.6000x; 1.3349x over previous
import functools

import jax
import jax.numpy as jnp
from jax import lax
from jax.experimental import pallas as pl
from jax.experimental.pallas import tpu as pltpu
from jax.experimental.pallas import tpu_sc as plsc

N_NODES = 10000
N_EDGES = 320000
N_GRAPHS = 64
D_IN, D_HID, D_OUT = 128, 256, 128

NC, NS = 2, 16
LANES = 16
CHK = 80
ROWS_PER_TILE = N_NODES // NS

_MESH = plsc.VectorSubcoreMesh(
    core_axis_name="c", subcore_axis_name="s", num_cores=NC, num_subcores=NS)


def _relu_inplace(ref, n):
  def body(v, _):
    sl = pl.ds(v * LANES, LANES)
    ref[sl] = jnp.maximum(ref[sl], 0.0)
    return _

  lax.fori_loop(0, n // LANES, body, None)


def _make_deg_kernel():
  epw = N_EDGES // (NC * NS)
  nchk = epw // CHK
  npad = 10240
  rpt = npad // NS

  @functools.partial(
      pl.kernel,
      out_type=jax.ShapeDtypeStruct((NC, npad), jnp.float32),
      mesh=_MESH,
      scratch_types=[
          pltpu.VMEM((epw,), jnp.int32),
          pltpu.VMEM((epw,), jnp.float32),
          pltpu.VMEM((CHK,), jnp.int32),
          pltpu.VMEM((rpt,), jnp.float32),
          pltpu.VMEM_SHARED((npad,), jnp.float32),
      ],
      name="gcn_deg_sc",
  )
  def deg_kernel(col_hbm, ew_hbm, out_hbm, colflat, ewflat, colchunk, zbuf,
                 acc):
    cid = lax.axis_index("c")
    sid = lax.axis_index("s")
    estart = (cid * NS + sid) * epw
    pltpu.sync_copy(col_hbm.at[pl.ds(estart, epw)], colflat)
    pltpu.sync_copy(ew_hbm.at[pl.ds(estart, epw)], ewflat)
    _relu_inplace(ewflat, epw)

    def zrow(r, _):
      zbuf[pl.ds(r * LANES, LANES)] = jnp.zeros((LANES,), jnp.float32)
      return _

    lax.fori_loop(0, rpt // LANES, zrow, None)
    pltpu.sync_copy(zbuf, acc.at[pl.ds(sid * rpt, rpt)])
    plsc.subcore_barrier()

    def chunk(i, _):
      e0 = i * CHK
      for k in range(CHK // LANES):
        colchunk[pl.ds(k * LANES, LANES)] = colflat[pl.ds(e0 + k * LANES,
                                                          LANES)]
      pltpu.sync_copy(ewflat.at[pl.ds(e0, CHK)], acc.at[colchunk], add=True)
      return _

    lax.fori_loop(0, nchk, chunk, None)
    plsc.subcore_barrier()
    pltpu.sync_copy(acc.at[pl.ds(sid * rpt, rpt)],
                    out_hbm.at[cid, pl.ds(sid * rpt, rpt)])
    return None

  return deg_kernel


def _make_spmm_kernel(split_edges, ntab):
  epw = N_EDGES // (NS * (NC if split_edges else 1))
  blk = 2000
  npad = 10240
  rpt = npad // NS

  @functools.partial(
      pl.kernel,
      out_type=jax.ShapeDtypeStruct((NC, npad, D_OUT), jnp.float32),
      mesh=_MESH,
      scratch_types=[
          pltpu.VMEM((blk,), jnp.int32),
          pltpu.VMEM((blk,), jnp.int32),
          pltpu.VMEM((blk,), jnp.float32),
          pltpu.VMEM((CHK,), jnp.int32),
          pltpu.VMEM((CHK,), jnp.int32),
          pltpu.VMEM((CHK,), jnp.int32),
          pltpu.VMEM((CHK, D_OUT), jnp.float32),
          pltpu.VMEM((CHK, D_OUT), jnp.float32),
          pltpu.VMEM((CHK, D_OUT), jnp.float32),
          pltpu.VMEM_SHARED((npad, D_OUT), jnp.float32),
          pltpu.SemaphoreType.DMA,
          pltpu.SemaphoreType.DMA,
          pltpu.SemaphoreType.DMA,
          pltpu.SemaphoreType.DMA,
          pltpu.SemaphoreType.DMA,
          pltpu.SemaphoreType.DMA,
      ],
      name="gcn_spmm_sc",
  )
  def spmm_kernel(row_hbm, col_hbm, ew_hbm, tbl_hbm, out_hbm, rowblk,
                  colblk, ewblk, cc0, cc1, cc2, gbuf0, gbuf1, gbuf2, acc,
                  gs0, gs1, gs2, ss0, ss1, ss2):
    cid = lax.axis_index("c")
    sid = lax.axis_index("s")
    if split_edges:
      estart = (cid * NS + sid) * epw
    else:
      estart = sid * epw

    def zrow(r, _):
      for k in range(D_OUT // LANES):
        gbuf0[r, pl.ds(k * LANES, LANES)] = jnp.zeros((LANES,), jnp.float32)
      return _

    lax.fori_loop(0, CHK, zrow, None)
    for k in range(rpt // CHK):
      pltpu.sync_copy(gbuf0, acc.at[pl.ds(sid * rpt + k * CHK, CHK)])
    plsc.subcore_barrier()

    nchunk = blk // CHK
    assert nchunk % 3 == 1
    last = nchunk - 1

    def gather(ci, gbuf, sem):
      ci = jnp.minimum(ci, last)
      return pltpu.async_copy(tbl_hbm.at[rowblk.at[pl.ds(ci * CHK, CHK)]],
                              gbuf, sem)

    def scale(ci, gbuf, cc):
      e0 = ci * CHK
      for k in range(CHK // LANES):
        cc[pl.ds(k * LANES, LANES)] = colblk[pl.ds(e0 + k * LANES, LANES)]

      def rowgrp(jb, c):
        ewv = jnp.maximum(ewblk[pl.ds(e0 + jb * LANES, LANES)], 0.0)
        for j2 in range(LANES):
          sv = jnp.full((LANES,), ewv[j2], jnp.float32)
          j = jb * LANES + j2
          for k in range(D_OUT // LANES):
            sl = pl.ds(k * LANES, LANES)
            gbuf[j, sl] = gbuf[j, sl] * sv
        return c

      lax.fori_loop(0, CHK // LANES, rowgrp, None)

    def scatter(gbuf, cc, sem):
      return pltpu.async_copy(gbuf, acc.at[cc], sem, add=True)

    def block(bi, _):
      eb = estart + bi * blk
      pltpu.sync_copy(row_hbm.at[pl.ds(eb, blk)], rowblk)
      pltpu.sync_copy(col_hbm.at[pl.ds(eb, blk)], colblk)
      pltpu.sync_copy(ew_hbm.at[pl.ds(eb, blk)], ewblk)

      if ntab == 2:
        def addoff(v, c):
          sl = pl.ds(v * LANES, LANES)
          rowblk[sl] = rowblk[sl] + cid * N_NODES
          return c

        lax.fori_loop(0, blk // LANES, addoff, None)

      g1 = gather(0, gbuf0, gs0)
      gather(1, gbuf1, gs1)
      g1.wait()

      def triple(ti, c):
        c0 = ti * 3
        g2 = gather(c0 + 2, gbuf2, gs2)
        scale(c0, gbuf0, cc0)
        s0 = scatter(gbuf0, cc0, ss0)
        pltpu.make_async_copy(
            tbl_hbm.at[rowblk.at[pl.ds(0, CHK)]], gbuf1, gs1).wait()
        scale(c0 + 1, gbuf1, cc1)
        s1 = scatter(gbuf1, cc1, ss1)
        s0.wait()
        g0 = gather(c0 + 3, gbuf0, gs0)
        g2.wait()
        scale(c0 + 2, gbuf2, cc2)
        s2 = scatter(gbuf2, cc2, ss2)
        s1.wait()
        gather(c0 + 4, gbuf1, gs1)
        g0.wait()
        s2.wait()
        return c

      lax.fori_loop(0, nchunk // 3, triple, None)
      scale(last, gbuf0, cc0)
      pltpu.sync_copy(gbuf0, acc.at[cc0], add=True)
      pltpu.make_async_copy(
          tbl_hbm.at[rowblk.at[pl.ds(0, CHK)]], gbuf1, gs1).wait()
      return _

    lax.fori_loop(0, epw // blk, block, None)
    plsc.subcore_barrier()
    pltpu.sync_copy(acc.at[pl.ds(sid * rpt, rpt)],
                    out_hbm.at[cid, pl.ds(sid * rpt, rpt)])
    return None

  return spmm_kernel


_deg_kernel = _make_deg_kernel()
_spmm = _make_spmm_kernel(split_edges=True, ntab=1)

NB = 1000
NBLK = N_NODES // NB


def _dis_block(degp):
  deg = degp[:, 0:1] + degp[:, 1:2] + 1.0
  return lax.rsqrt(deg)


def _tca_body(degp_ref, x_ref, xs_ref):
  dis = _dis_block(degp_ref[...])
  xs_ref[...] = x_ref[...] * dis


def _tcb_body(degp_ref, s1_ref, xs_ref, b1_ref, w1_ref, w2_ref, g2_ref):
  dis = _dis_block(degp_ref[...])
  t = s1_ref[0] + s1_ref[1] + xs_ref[...]
  hr = jnp.maximum(
      jnp.dot(t, w1_ref[...], preferred_element_type=jnp.float32) * dis +
      b1_ref[...], 0.0)
  h2 = jnp.dot(hr, w2_ref[...], preferred_element_type=jnp.float32)
  g2_ref[...] = h2 * dis


def _tcc_body(degp_ref, s2_ref, g2_ref, b2_ref, batch_ref, wm_ref, bm_ref,
              out_ref, pooled, counts):
  i = pl.program_id(0)
  dis = _dis_block(degp_ref[...])
  hf = jnp.maximum((s2_ref[0] + s2_ref[1] + g2_ref[...]) * dis +
                   b2_ref[0:1, :], 0.0)
  onehot = (batch_ref[...] == lax.broadcasted_iota(
      jnp.int32, (1, N_GRAPHS), 1)).astype(jnp.float32)
  psum = lax.dot_general(onehot, hf, (((0,), (0,)), ((), ())),
                         preferred_element_type=jnp.float32)
  csum = lax.dot_general(onehot, jnp.ones((NB, 1), jnp.float32),
                         (((0,), (0,)), ((), ())),
                         preferred_element_type=jnp.float32)

  @pl.when(i == 0)
  def _():
    pooled[...] = jnp.zeros_like(pooled)
    counts[...] = jnp.zeros_like(counts)

  pooled[...] += psum
  counts[...] += csum
  mean = pooled[...] / jnp.maximum(counts[...], 1.0)
  out_ref[...] = jnp.dot(mean, wm_ref[...],
                         preferred_element_type=jnp.float32) + bm_ref[...]


def kernel(x, edge_index, edge_weight, batch, W1, b1, W2, b2, Wm, bm):
  row = edge_index[0].astype(jnp.int32)
  col = edge_index[1].astype(jnp.int32)
  ew = edge_weight.astype(jnp.float32)

  degp = _deg_kernel(col, ew)[:, :N_NODES]
  degp_t = degp.T

  xs = pl.pallas_call(
      _tca_body,
      grid=(NBLK,),
      in_specs=[
          pl.BlockSpec((NB, 2), lambda i: (i, 0)),
          pl.BlockSpec((NB, D_IN), lambda i: (i, 0)),
      ],
      out_specs=pl.BlockSpec((NB, D_IN), lambda i: (i, 0)),
      out_shape=jax.ShapeDtypeStruct((N_NODES, D_IN), jnp.float32),
      name="gcn_tca",
  )(degp_t, x)

  s1 = _spmm(row, col, ew, xs)[:, :N_NODES]

  g2 = pl.pallas_call(
      _tcb_body,
      grid=(NBLK,),
      in_specs=[
          pl.BlockSpec((NB, 2), lambda i: (i, 0)),
          pl.BlockSpec((2, NB, 128), lambda i: (0, i, 0)),
          pl.BlockSpec((NB, D_IN), lambda i: (i, 0)),
          pl.BlockSpec((1, D_HID), lambda i: (0, 0)),
          pl.BlockSpec((D_IN, D_HID), lambda i: (0, 0)),
          pl.BlockSpec((D_HID, D_OUT), lambda i: (0, 0)),
      ],
      out_specs=pl.BlockSpec((NB, D_OUT), lambda i: (i, 0)),
      out_shape=jax.ShapeDtypeStruct((N_NODES, D_OUT), jnp.float32),
      name="gcn_tcb",
  )(degp_t, s1, xs, b1.reshape(1, D_HID), W1, W2)

  s2 = _spmm(row, col, ew, g2)[:, :N_NODES]

  out = pl.pallas_call(
      _tcc_body,
      grid=(NBLK,),
      in_specs=[
          pl.BlockSpec((NB, 2), lambda i: (i, 0)),
          pl.BlockSpec((2, NB, D_OUT), lambda i: (0, i, 0)),
          pl.BlockSpec((NB, D_OUT), lambda i: (i, 0)),
          pl.BlockSpec((1, D_OUT), lambda i: (0, 0)),
          pl.BlockSpec((NB, 1), lambda i: (i, 0)),
          pl.BlockSpec((D_OUT, 2), lambda i: (0, 0)),
          pl.BlockSpec((1, 2), lambda i: (0, 0)),
      ],
      out_specs=pl.BlockSpec((N_GRAPHS, 2), lambda i: (0, 0)),
      out_shape=jax.ShapeDtypeStruct((N_GRAPHS, 2), jnp.float32),
      scratch_shapes=[
          pltpu.VMEM((N_GRAPHS, D_OUT), jnp.float32),
          pltpu.VMEM((N_GRAPHS, 1), jnp.float32),
      ],
      name="gcn_tcc",
  )(degp_t, s2, g2, b2.reshape(1, D_OUT), batch.astype(jnp.int32)[:, None],
    Wm, bm.reshape(1, 2))
  return out
